# baseline (speedup 1.0000x reference)
import functools

import jax
import jax.numpy as jnp
from jax import lax
from jax.experimental import pallas as pl
from jax.experimental.pallas import tpu as pltpu
from jax.experimental.pallas import tpu_sc as plsc

_N = 10000
_D = 256
_E = 160000
_K = 4

_NC = 2
_NS = 16
_NW = _NC * _NS
_L = 16

_NP = 320
_NPAD = _NW * _NP
_NDEG = 10240
_SEG = _NDEG // _NS
_CAP = 8192
_ECHUNK = 2000
_G = 128
_LB = 1024

_ROW_TILE = 1000


def _pack_bf16_tc(lo, hi):
    ilo = lax.bitcast_convert_type(lo, jnp.int32)
    ihi = lax.bitcast_convert_type(hi, jnp.int32)
    wlo = lax.shift_right_logical(ilo + 0x8000, 16)
    whi = (ihi + 0x8000) & (-65536)
    return whi | wlo


def _rownorm_body(x_ref, d_ref, o_ref, g_ref):
    x = x_ref[...]
    n = jnp.sqrt(jnp.sum(x * x, axis=1, keepdims=True))
    h = x / jnp.maximum(n, 1e-12)
    o_ref[...] = h
    g = h * d_ref[...]
    g_ref[...] = _pack_bf16_tc(g[:, : _D // 2], g[:, _D // 2 :])


def _colsumsq_body(h0, h1, h2, h3, h4, o_ref):
    @pl.when(pl.program_id(0) == 0)
    def _():
        o_ref[...] = jnp.zeros_like(o_ref)
    for i, h in enumerate((h0, h1, h2, h3, h4)):
        x = h[...]
        o_ref[i, :, :] += jnp.sum(x * x, axis=0, keepdims=True)


def _colscale_body(h0, h1, h2, h3, h4, s_ref, o_ref):
    for i, h in enumerate((h0, h1, h2, h3, h4)):
        o_ref[i, :, :] = h[...] * s_ref[i]


def _row_normalize(xp, dis):
    return pl.pallas_call(
        _rownorm_body,
        grid=(_NPAD // 1024,),
        in_specs=[pl.BlockSpec((1024, _D), lambda i: (i, 0)),
                  pl.BlockSpec((1024, 1), lambda i: (i, 0))],
        out_specs=(pl.BlockSpec((1024, _D), lambda i: (i, 0)),
                   pl.BlockSpec((1024, _D // 2), lambda i: (i, 0))),
        out_shape=(jax.ShapeDtypeStruct((_NPAD, _D), jnp.float32),
                   jax.ShapeDtypeStruct((_NPAD, _D // 2), jnp.int32)),
    )(xp, dis.reshape(_NPAD, 1))


def _col_normalize(lxs):
    hspec = pl.BlockSpec((_ROW_TILE, _D), lambda j: (j, 0))
    sumsq = pl.pallas_call(
        _colsumsq_body,
        grid=(_N // _ROW_TILE,),
        in_specs=[hspec] * 5,
        out_specs=pl.BlockSpec((5, 1, _D), lambda j: (0, 0, 0)),
        out_shape=jax.ShapeDtypeStruct((5, 1, _D), jnp.float32),
    )(*lxs)
    scale = 1.0 / jnp.maximum(jnp.sqrt(sumsq), 1e-12)
    return pl.pallas_call(
        _colscale_body,
        grid=(_N // _ROW_TILE,),
        in_specs=[hspec] * 5 + [pl.BlockSpec((5, 1, _D), lambda j: (0, 0, 0))],
        out_specs=pl.BlockSpec((5, _ROW_TILE, _D), lambda j: (0, j, 0)),
        out_shape=jax.ShapeDtypeStruct((5, _N, _D), jnp.float32),
    )(*lxs, scale)


_sc_mesh = plsc.VectorSubcoreMesh(
    core_axis_name="c", subcore_axis_name="s",
    num_cores=_NC, num_subcores=_NS)


def _edge_prep_body(row_hbm, col_hbm, srcs_hbm, dsts_hbm, cnts_hbm, dis_hbm,
                    degb, tmpb, tmpb2, srcl, dstl, rbuf0, cbuf0, rbuf1, cbuf1,
                    cntb, dislb, shdeg, shdis, sem0, sem1):
    cid = lax.axis_index("c")
    sid = lax.axis_index("s")
    wid = cid * _NS + sid
    lo = wid * _NP
    iota = lax.iota(jnp.int32, _L)
    zf = jnp.zeros((_L,), jnp.float32)
    zi = jnp.zeros((_L,), jnp.int32)

    def _z(i, c):
        degb[pl.ds(i * _L, _L)] = zf
        return c
    lax.fori_loop(0, _NDEG // _L, _z, 0)

    ebase = sid * (_E // _NS)
    for ch in range(_E // _NS // _ECHUNK):
        pltpu.sync_copy(row_hbm.at[pl.ds(ebase + ch * _ECHUNK, _ECHUNK)],
                        rbuf0)
        pltpu.sync_copy(col_hbm.at[pl.ds(ebase + ch * _ECHUNK, _ECHUNK)],
                        cbuf0)

        def _deg(v, c):
            r = rbuf0[pl.ds(v * _L, _L)]
            cc = cbuf0[pl.ds(v * _L, _L)]
            w = jnp.where(r != cc, 1.0, 0.0).astype(jnp.float32)
            plsc.addupdate_scatter(degb, [r], w)
            return c
        lax.fori_loop(0, _ECHUNK // _L, _deg, 0)

    pltpu.sync_copy(degb, shdeg.at[sid])
    plsc.subcore_barrier()
    pltpu.sync_copy(shdeg.at[0, pl.ds(sid * _SEG, _SEG)], tmpb)
    for t in range(1, _NS):
        pltpu.sync_copy(shdeg.at[t, pl.ds(sid * _SEG, _SEG)], tmpb2)

        def _acc(v, c):
            s = pl.ds(v * _L, _L)
            tmpb[s] = tmpb[s] + tmpb2[s]
            return c
        lax.fori_loop(0, _SEG // _L, _acc, 0)

    def _newton(v, c):
        s = pl.ds(v * _L, _L)
        d = tmpb[s]
        x = jnp.maximum(d, 1.0)
        i = plsc.bitcast(x, jnp.int32)
        i = jnp.int32(0x5F3759DF) - lax.shift_right_logical(i, 1)
        y = plsc.bitcast(i, jnp.float32)
        for _ in range(3):
            y = y * (1.5 - 0.5 * x * y * y)
        tmpb[s] = jnp.where(d > 0.0, y, 0.0)
        return c
    lax.fori_loop(0, _SEG // _L, _newton, 0)
    pltpu.sync_copy(tmpb, shdis.at[pl.ds(sid * _SEG, _SEG)])
    plsc.subcore_barrier()

    pltpu.sync_copy(shdis.at[pl.ds(lo, _NP)], dislb)
    pltpu.sync_copy(dislb, dis_hbm.at[pl.ds(lo, _NP)])

    def _scan_buf(rbuf, cbuf, basev):
        @plsc.parallel_loop(0, _ECHUNK // _L, unroll=2, carry=basev)
        def _scan(v, bv):
            r = rbuf[pl.ds(v * _L, _L)]
            cc = cbuf[pl.ds(v * _L, _L)]
            m = (cc >= lo) & (cc < lo + _NP) & (r != cc)
            mi = jnp.where(m, 1, 0).astype(jnp.int32)
            cs = plsc.cumsum(mi)
            pos = bv + cs - mi
            mok = m & (pos < _CAP)
            plsc.store_scatter(srcl, [pos], r, mask=mok)
            plsc.store_scatter(dstl, [pos],
                               lax.shift_left(cc - lo, 8), mask=mok)
            return jnp.minimum(bv + jnp.max(cs), _CAP)
        return _scan

    nch = _E // _ECHUNK
    pltpu.async_copy(row_hbm.at[pl.ds(0, _ECHUNK)], rbuf0, sem0)
    pltpu.async_copy(col_hbm.at[pl.ds(0, _ECHUNK)], cbuf0, sem0)

    def _pair(p, basev):
        c0 = 2 * p
        pltpu.make_async_copy(
            row_hbm.at[pl.ds(c0 * _ECHUNK, _ECHUNK)], rbuf0, sem0).wait()
        pltpu.make_async_copy(
            col_hbm.at[pl.ds(c0 * _ECHUNK, _ECHUNK)], cbuf0, sem0).wait()
        pltpu.async_copy(
            row_hbm.at[pl.ds((c0 + 1) * _ECHUNK, _ECHUNK)], rbuf1, sem1)
        pltpu.async_copy(
            col_hbm.at[pl.ds((c0 + 1) * _ECHUNK, _ECHUNK)], cbuf1, sem1)
        basev = _scan_buf(rbuf0, cbuf0, basev)
        pltpu.make_async_copy(
            row_hbm.at[pl.ds((c0 + 1) * _ECHUNK, _ECHUNK)], rbuf1,
            sem1).wait()
        pltpu.make_async_copy(
            col_hbm.at[pl.ds((c0 + 1) * _ECHUNK, _ECHUNK)], cbuf1,
            sem1).wait()

        @pl.when(p + 1 < nch // 2)
        def _():
            pltpu.async_copy(
                row_hbm.at[pl.ds((c0 + 2) * _ECHUNK, _ECHUNK)], rbuf0, sem0)
            pltpu.async_copy(
                col_hbm.at[pl.ds((c0 + 2) * _ECHUNK, _ECHUNK)], cbuf0, sem0)
        basev = _scan_buf(rbuf1, cbuf1, basev)
        return basev

    basev = lax.fori_loop(0, nch // 2, _pair, jnp.zeros((_L,), jnp.int32))

    trash = jnp.full((_L,), _NP * _D, jnp.int32)
    for j in range(_G // _L):
        pos = basev + j * _L + iota
        mok = pos < _CAP
        plsc.store_scatter(srcl, [pos], zi, mask=mok)
        plsc.store_scatter(dstl, [pos], trash, mask=mok)

    cnt = jnp.minimum(basev, _CAP - _G)
    for j in range(128 // _L):
        cntb[pl.ds(j * _L, _L)] = cnt
    pltpu.sync_copy(cntb, cnts_hbm.at[wid])
    pltpu.sync_copy(srcl, srcs_hbm.at[wid])
    pltpu.sync_copy(dstl, dsts_hbm.at[wid])


_edge_prep = functools.partial(
    pl.kernel,
    out_type=(
        jax.ShapeDtypeStruct((_NW, _CAP), jnp.int32),
        jax.ShapeDtypeStruct((_NW, _CAP), jnp.int32),
        jax.ShapeDtypeStruct((_NW, 128), jnp.int32),
        jax.ShapeDtypeStruct((_NPAD,), jnp.float32),
    ),
    mesh=_sc_mesh,
    compiler_params=pltpu.CompilerParams(needs_layout_passes=False),
    scratch_types=[
        pltpu.VMEM((_NDEG,), jnp.float32),
        pltpu.VMEM((_SEG,), jnp.float32),
        pltpu.VMEM((_SEG,), jnp.float32),
        pltpu.VMEM((_CAP,), jnp.int32),
        pltpu.VMEM((_CAP,), jnp.int32),
        pltpu.VMEM((_ECHUNK,), jnp.int32),
        pltpu.VMEM((_ECHUNK,), jnp.int32),
        pltpu.VMEM((_ECHUNK,), jnp.int32),
        pltpu.VMEM((_ECHUNK,), jnp.int32),
        pltpu.VMEM((128,), jnp.int32),
        pltpu.VMEM((_NP,), jnp.float32),
        pltpu.VMEM_SHARED((_NS, _NDEG), jnp.float32),
        pltpu.VMEM_SHARED((_NDEG,), jnp.float32),
        pltpu.SemaphoreType.DMA,
        pltpu.SemaphoreType.DMA,
    ],
)(_edge_prep_body)


def _propagate_body(emit_g, h_hbm, g_hbm, srcs_hbm, dsts_hbm, cnts_hbm,
                    dis_hbm, *refs):
    if emit_g:
        (hn_hbm, gn_hbm, agg, rows0, rows1, hstage, sbuf, dbuf, cbuf, dislb,
         sem) = refs
    else:
        (hn_hbm, agg, rows0, rows1, hstage, sbuf, dbuf, cbuf, dislb,
         sem) = refs
        gn_hbm = None
    cid = lax.axis_index("c")
    sid = lax.axis_index("s")
    wid = cid * _NS + sid
    lo = wid * _NP
    iota = lax.iota(jnp.int32, _L)
    zf = jnp.zeros((_L,), jnp.float32)
    hmask = jnp.full((_L,), -65536, jnp.int32)
    half = jnp.full((_L,), 0x8000, jnp.int32)

    @plsc.parallel_loop(0, (_NP + 1) * _D // _L, unroll=4)
    def _z(i):
        agg[pl.ds(i * _L, _L)] = zf

    pltpu.sync_copy(cnts_hbm.at[wid], cbuf)
    pltpu.sync_copy(dis_hbm.at[pl.ds(lo, _NP)], dislb)
    cnt = jnp.max(cbuf[pl.ds(0, _L)])
    nouter = (cnt + _LB - 1) // _LB

    def _accum(rbufref, g):
        @plsc.parallel_loop(0, _G, unroll=2)
        def _e(e):
            v = g * _G + e
            vf = jnp.full((_L,), v, jnp.int32)
            dstb = plsc.load_gather(dbuf, [vf])
            for j in range(_D // 2 // _L):
                w = rbufref[e, pl.ds(j * _L, _L)]
                flo = plsc.bitcast(lax.shift_left(w, 16), jnp.float32)
                fhi = plsc.bitcast(w & hmask, jnp.float32)
                plsc.addupdate_scatter(
                    agg, [dstb + (iota + j * _L)], flo)
                plsc.addupdate_scatter(
                    agg, [dstb + (iota + (_D // 2 + j * _L))], fhi)

    def _outer(b, c):
        pltpu.sync_copy(srcs_hbm.at[wid, pl.ds(b * _LB, _LB)], sbuf)
        pltpu.sync_copy(dsts_hbm.at[wid, pl.ds(b * _LB, _LB)], dbuf)
        rem = jnp.minimum(cnt - b * _LB, _LB)
        ng = (rem + _G - 1) // _G
        pltpu.async_copy(g_hbm.at[sbuf.at[pl.ds(0, _G)]], rows0, sem)

        def _gath(g, c2):
            @pl.when(g % 2 == 0)
            def _():
                pltpu.make_async_copy(
                    g_hbm.at[sbuf.at[pl.ds(0, _G)]], rows0, sem).wait()

                @pl.when(g + 1 < ng)
                def _():
                    pltpu.async_copy(
                        g_hbm.at[sbuf.at[pl.ds((g + 1) * _G, _G)]], rows1,
                        sem)
                _accum(rows0, g)

            @pl.when(g % 2 == 1)
            def _():
                pltpu.make_async_copy(
                    g_hbm.at[sbuf.at[pl.ds(0, _G)]], rows1, sem).wait()

                @pl.when(g + 1 < ng)
                def _():
                    pltpu.async_copy(
                        g_hbm.at[sbuf.at[pl.ds((g + 1) * _G, _G)]], rows0,
                        sem)
                _accum(rows1, g)
            return c2
        lax.fori_loop(0, ng, _gath, 0)
        return c
    lax.fori_loop(0, nouter, _outer, 0)

    off = 0
    for nrows in (32,) * (_NP // 32):
        pltpu.sync_copy(h_hbm.at[pl.ds(lo + off, nrows)],
                        hstage.at[pl.ds(0, nrows)])

        @plsc.parallel_loop(0, nrows, unroll=2)
        def _fin(i):
            dv = plsc.load_gather(dislb, [jnp.full((_L,), off + i,
                                                   jnp.int32)])
            for j in range(_D // 2 // _L):
                sl = pl.ds(j * _L, _L)
                sh = pl.ds(_D // 2 + j * _L, _L)
                a0 = agg[pl.ds((off + i) * _D + j * _L, _L)]
                a1 = agg[pl.ds((off + i) * _D + _D // 2 + j * _L, _L)]
                hn0 = -dv * a0 - hstage[i, sl]
                hn1 = -dv * a1 - hstage[i, sh]
                hstage[i, sl] = hn0
                hstage[i, sh] = hn1
                if emit_g:
                    i0 = plsc.bitcast(dv * hn0, jnp.int32) + half
                    i1 = plsc.bitcast(dv * hn1, jnp.int32) + half
                    rows0[i, sl] = ((i1 & hmask)
                                    | lax.shift_right_logical(i0, 16))
        pltpu.sync_copy(hstage.at[pl.ds(0, nrows)],
                        hn_hbm.at[pl.ds(lo + off, nrows)])
        if emit_g:
            pltpu.sync_copy(rows0.at[pl.ds(0, nrows)],
                            gn_hbm.at[pl.ds(lo + off, nrows)])
        off += nrows


_prop_scratch = [
    pltpu.VMEM(((_NP + 1) * _D,), jnp.float32),
    pltpu.VMEM((_G, _D // 2), jnp.int32),
    pltpu.VMEM((_G, _D // 2), jnp.int32),
    pltpu.VMEM((32, _D), jnp.float32),
    pltpu.VMEM((_LB,), jnp.int32),
    pltpu.VMEM((_LB,), jnp.int32),
    pltpu.VMEM((128,), jnp.int32),
    pltpu.VMEM((_NP,), jnp.float32),
    pltpu.SemaphoreType.DMA,
]

_propagate = functools.partial(
    pl.kernel,
    out_type=(
        jax.ShapeDtypeStruct((_NPAD, _D), jnp.float32),
        jax.ShapeDtypeStruct((_NPAD, _D // 2), jnp.int32),
    ),
    mesh=_sc_mesh,
    compiler_params=pltpu.CompilerParams(needs_layout_passes=False),
    scratch_types=_prop_scratch,
)(functools.partial(_propagate_body, True))

_propagate_last = functools.partial(
    pl.kernel,
    out_type=jax.ShapeDtypeStruct((_NPAD, _D), jnp.float32),
    mesh=_sc_mesh,
    compiler_params=pltpu.CompilerParams(needs_layout_passes=False),
    scratch_types=_prop_scratch,
)(functools.partial(_propagate_body, False))


def kernel(x, edge_index):
    row = edge_index[0]
    col = edge_index[1]
    srcs, dsts, cnts, dis = _edge_prep(row, col)
    xp = jnp.concatenate(
        [x, jnp.zeros((_NPAD - _N, _D), jnp.float32)], axis=0)
    h, g = _row_normalize(xp, dis)
    lxs = [h]
    for k in range(_K):
        if k < _K - 1:
            h, g = _propagate(h, g, srcs, dsts, cnts, dis)
        else:
            h = _propagate_last(h, g, srcs, dsts, cnts, dis)
        lxs.append(h)
    return _col_normalize(lxs)


# speedup vs baseline: 1.2053x; 1.2053x over previous
import functools

import jax
import jax.numpy as jnp
from jax import lax
from jax.experimental import pallas as pl
from jax.experimental.pallas import tpu as pltpu
from jax.experimental.pallas import tpu_sc as plsc

_N = 10000
_D = 256
_E = 160000
_K = 4

_NC = 2
_NS = 16
_NW = _NC * _NS
_L = 16

_NP = 320
_NPAD = _NW * _NP
_NDEG = 10240
_SEG = _NDEG // _NS
_CAP = 8192
_ECHUNK = 2000
_G = 64
_LB = 1024

_ROW_TILE = 1000


def _pack_bf16_tc(lo, hi):
    ilo = lax.bitcast_convert_type(lo, jnp.int32)
    ihi = lax.bitcast_convert_type(hi, jnp.int32)
    wlo = lax.shift_right_logical(ilo + 0x8000, 16)
    whi = (ihi + 0x8000) & (-65536)
    return whi | wlo


def _rownorm_body(x_ref, d_ref, o_ref, g_ref):
    x = x_ref[...]
    n = jnp.sqrt(jnp.sum(x * x, axis=1, keepdims=True))
    h = x / jnp.maximum(n, 1e-12)
    o_ref[...] = h
    g = h * d_ref[...]
    g_ref[...] = _pack_bf16_tc(g[:, : _D // 2], g[:, _D // 2 :])


def _colsumsq_body(h0, h1, h2, h3, h4, o_ref):
    @pl.when(pl.program_id(0) == 0)
    def _():
        o_ref[...] = jnp.zeros_like(o_ref)
    for i, h in enumerate((h0, h1, h2, h3, h4)):
        x = h[...]
        o_ref[i, :, :] += jnp.sum(x * x, axis=0, keepdims=True)


def _colscale_body(h0, h1, h2, h3, h4, s_ref, o_ref):
    for i, h in enumerate((h0, h1, h2, h3, h4)):
        o_ref[i, :, :] = h[...] * s_ref[i]


def _row_normalize(xp, dis):
    return pl.pallas_call(
        _rownorm_body,
        grid=(_NPAD // 1024,),
        in_specs=[pl.BlockSpec((1024, _D), lambda i: (i, 0)),
                  pl.BlockSpec((1024, 1), lambda i: (i, 0))],
        out_specs=(pl.BlockSpec((1024, _D), lambda i: (i, 0)),
                   pl.BlockSpec((1024, _D // 2), lambda i: (i, 0))),
        out_shape=(jax.ShapeDtypeStruct((_NPAD, _D), jnp.float32),
                   jax.ShapeDtypeStruct((_NPAD, _D // 2), jnp.int32)),
    )(xp, dis.reshape(_NPAD, 1))


def _col_normalize(lxs):
    hspec = pl.BlockSpec((_ROW_TILE, _D), lambda j: (j, 0))
    sumsq = pl.pallas_call(
        _colsumsq_body,
        grid=(_N // _ROW_TILE,),
        in_specs=[hspec] * 5,
        out_specs=pl.BlockSpec((5, 1, _D), lambda j: (0, 0, 0)),
        out_shape=jax.ShapeDtypeStruct((5, 1, _D), jnp.float32),
    )(*lxs)
    scale = 1.0 / jnp.maximum(jnp.sqrt(sumsq), 1e-12)
    return pl.pallas_call(
        _colscale_body,
        grid=(_N // _ROW_TILE,),
        in_specs=[hspec] * 5 + [pl.BlockSpec((5, 1, _D), lambda j: (0, 0, 0))],
        out_specs=pl.BlockSpec((5, _ROW_TILE, _D), lambda j: (0, j, 0)),
        out_shape=jax.ShapeDtypeStruct((5, _N, _D), jnp.float32),
    )(*lxs, scale)


_sc_mesh = plsc.VectorSubcoreMesh(
    core_axis_name="c", subcore_axis_name="s",
    num_cores=_NC, num_subcores=_NS)


def _edge_prep_body(row_hbm, col_hbm, srcs_hbm, dsts_hbm, cnts_hbm, dis_hbm,
                    degb, tmpb, tmpb2, srcl, dstl, rbuf0, cbuf0, rbuf1, cbuf1,
                    cntb, dislb, shdeg, shdis, sem0, sem1):
    cid = lax.axis_index("c")
    sid = lax.axis_index("s")
    wid = cid * _NS + sid
    lo = wid * _NP
    iota = lax.iota(jnp.int32, _L)
    zf = jnp.zeros((_L,), jnp.float32)
    zi = jnp.zeros((_L,), jnp.int32)

    def _z(i, c):
        degb[pl.ds(i * _L, _L)] = zf
        return c
    lax.fori_loop(0, _NDEG // _L, _z, 0)

    ebase = sid * (_E // _NS)
    for ch in range(_E // _NS // _ECHUNK):
        pltpu.sync_copy(row_hbm.at[pl.ds(ebase + ch * _ECHUNK, _ECHUNK)],
                        rbuf0)
        pltpu.sync_copy(col_hbm.at[pl.ds(ebase + ch * _ECHUNK, _ECHUNK)],
                        cbuf0)

        def _deg(v, c):
            r = rbuf0[pl.ds(v * _L, _L)]
            cc = cbuf0[pl.ds(v * _L, _L)]
            w = jnp.where(r != cc, 1.0, 0.0).astype(jnp.float32)
            plsc.addupdate_scatter(degb, [r], w)
            return c
        lax.fori_loop(0, _ECHUNK // _L, _deg, 0)

    pltpu.sync_copy(degb, shdeg.at[sid])
    plsc.subcore_barrier()
    pltpu.sync_copy(shdeg.at[0, pl.ds(sid * _SEG, _SEG)], tmpb)
    for t in range(1, _NS):
        pltpu.sync_copy(shdeg.at[t, pl.ds(sid * _SEG, _SEG)], tmpb2)

        def _acc(v, c):
            s = pl.ds(v * _L, _L)
            tmpb[s] = tmpb[s] + tmpb2[s]
            return c
        lax.fori_loop(0, _SEG // _L, _acc, 0)

    def _newton(v, c):
        s = pl.ds(v * _L, _L)
        d = tmpb[s]
        x = jnp.maximum(d, 1.0)
        i = plsc.bitcast(x, jnp.int32)
        i = jnp.int32(0x5F3759DF) - lax.shift_right_logical(i, 1)
        y = plsc.bitcast(i, jnp.float32)
        for _ in range(3):
            y = y * (1.5 - 0.5 * x * y * y)
        tmpb[s] = jnp.where(d > 0.0, y, 0.0)
        return c
    lax.fori_loop(0, _SEG // _L, _newton, 0)
    pltpu.sync_copy(tmpb, shdis.at[pl.ds(sid * _SEG, _SEG)])
    plsc.subcore_barrier()

    pltpu.sync_copy(shdis.at[pl.ds(lo, _NP)], dislb)
    pltpu.sync_copy(dislb, dis_hbm.at[pl.ds(lo, _NP)])

    def _scan_buf(rbuf, cbuf, basev):
        @plsc.parallel_loop(0, _ECHUNK // _L, unroll=2, carry=basev)
        def _scan(v, bv):
            r = rbuf[pl.ds(v * _L, _L)]
            cc = cbuf[pl.ds(v * _L, _L)]
            m = (cc >= lo) & (cc < lo + _NP) & (r != cc)
            mi = jnp.where(m, 1, 0).astype(jnp.int32)
            cs = plsc.cumsum(mi)
            pos = bv + cs - mi
            mok = m & (pos < _CAP)
            plsc.store_scatter(srcl, [pos], r, mask=mok)
            plsc.store_scatter(dstl, [pos],
                               lax.shift_left(cc - lo, 8), mask=mok)
            return jnp.minimum(bv + jnp.max(cs), _CAP)
        return _scan

    nch = _E // _ECHUNK
    pltpu.async_copy(row_hbm.at[pl.ds(0, _ECHUNK)], rbuf0, sem0)
    pltpu.async_copy(col_hbm.at[pl.ds(0, _ECHUNK)], cbuf0, sem0)

    def _pair(p, basev):
        c0 = 2 * p
        pltpu.make_async_copy(
            row_hbm.at[pl.ds(c0 * _ECHUNK, _ECHUNK)], rbuf0, sem0).wait()
        pltpu.make_async_copy(
            col_hbm.at[pl.ds(c0 * _ECHUNK, _ECHUNK)], cbuf0, sem0).wait()
        pltpu.async_copy(
            row_hbm.at[pl.ds((c0 + 1) * _ECHUNK, _ECHUNK)], rbuf1, sem1)
        pltpu.async_copy(
            col_hbm.at[pl.ds((c0 + 1) * _ECHUNK, _ECHUNK)], cbuf1, sem1)
        basev = _scan_buf(rbuf0, cbuf0, basev)
        pltpu.make_async_copy(
            row_hbm.at[pl.ds((c0 + 1) * _ECHUNK, _ECHUNK)], rbuf1,
            sem1).wait()
        pltpu.make_async_copy(
            col_hbm.at[pl.ds((c0 + 1) * _ECHUNK, _ECHUNK)], cbuf1,
            sem1).wait()

        @pl.when(p + 1 < nch // 2)
        def _():
            pltpu.async_copy(
                row_hbm.at[pl.ds((c0 + 2) * _ECHUNK, _ECHUNK)], rbuf0, sem0)
            pltpu.async_copy(
                col_hbm.at[pl.ds((c0 + 2) * _ECHUNK, _ECHUNK)], cbuf0, sem0)
        basev = _scan_buf(rbuf1, cbuf1, basev)
        return basev

    basev = lax.fori_loop(0, nch // 2, _pair, jnp.zeros((_L,), jnp.int32))

    trash = jnp.full((_L,), _NP * _D, jnp.int32)
    for j in range(_G // _L):
        pos = basev + j * _L + iota
        mok = pos < _CAP
        plsc.store_scatter(srcl, [pos], zi, mask=mok)
        plsc.store_scatter(dstl, [pos], trash, mask=mok)

    cnt = jnp.minimum(basev, _CAP - _G)
    for j in range(128 // _L):
        cntb[pl.ds(j * _L, _L)] = cnt
    pltpu.sync_copy(cntb, cnts_hbm.at[wid])
    pltpu.sync_copy(srcl, srcs_hbm.at[wid])
    pltpu.sync_copy(dstl, dsts_hbm.at[wid])


_edge_prep = functools.partial(
    pl.kernel,
    out_type=(
        jax.ShapeDtypeStruct((_NW, _CAP), jnp.int32),
        jax.ShapeDtypeStruct((_NW, _CAP), jnp.int32),
        jax.ShapeDtypeStruct((_NW, 128), jnp.int32),
        jax.ShapeDtypeStruct((_NPAD,), jnp.float32),
    ),
    mesh=_sc_mesh,
    compiler_params=pltpu.CompilerParams(needs_layout_passes=False),
    scratch_types=[
        pltpu.VMEM((_NDEG,), jnp.float32),
        pltpu.VMEM((_SEG,), jnp.float32),
        pltpu.VMEM((_SEG,), jnp.float32),
        pltpu.VMEM((_CAP,), jnp.int32),
        pltpu.VMEM((_CAP,), jnp.int32),
        pltpu.VMEM((_ECHUNK,), jnp.int32),
        pltpu.VMEM((_ECHUNK,), jnp.int32),
        pltpu.VMEM((_ECHUNK,), jnp.int32),
        pltpu.VMEM((_ECHUNK,), jnp.int32),
        pltpu.VMEM((128,), jnp.int32),
        pltpu.VMEM((_NP,), jnp.float32),
        pltpu.VMEM_SHARED((_NS, _NDEG), jnp.float32),
        pltpu.VMEM_SHARED((_NDEG,), jnp.float32),
        pltpu.SemaphoreType.DMA,
        pltpu.SemaphoreType.DMA,
    ],
)(_edge_prep_body)


def _propagate_body(emit_g, h_hbm, g_hbm, srcs_hbm, dsts_hbm, cnts_hbm,
                    dis_hbm, *refs):
    if emit_g:
        (hn_hbm, gn_hbm, agg, rows0, rows1, hstage, sbuf, dbuf, cbuf, dislb,
         sem) = refs
    else:
        (hn_hbm, agg, rows0, rows1, hstage, sbuf, dbuf, cbuf, dislb,
         sem) = refs
        gn_hbm = None
    cid = lax.axis_index("c")
    sid = lax.axis_index("s")
    wid = cid * _NS + sid
    lo = wid * _NP
    iota = lax.iota(jnp.int32, _L)
    zf = jnp.zeros((_L,), jnp.float32)
    hmask = jnp.full((_L,), -65536, jnp.int32)
    half = jnp.full((_L,), 0x8000, jnp.int32)

    @plsc.parallel_loop(0, (_NP + 1) * _D // _L, unroll=4)
    def _z(i):
        agg[pl.ds(i * _L, _L)] = zf

    pltpu.sync_copy(cnts_hbm.at[wid], cbuf)
    pltpu.sync_copy(dis_hbm.at[pl.ds(lo, _NP)], dislb)
    cnt = jnp.max(cbuf[pl.ds(0, _L)])
    nouter = (cnt + _LB - 1) // _LB

    def _accum(rbufref, g):
        @plsc.parallel_loop(0, _G, unroll=2)
        def _e(e):
            v = g * _G + e
            vf = jnp.full((_L,), v, jnp.int32)
            dstb = plsc.load_gather(dbuf, [vf])
            for j in range(_D // 2 // _L):
                w = rbufref[e, pl.ds(j * _L, _L)]
                flo = plsc.bitcast(lax.shift_left(w, 16), jnp.float32)
                fhi = plsc.bitcast(w & hmask, jnp.float32)
                plsc.addupdate_scatter(
                    agg, [dstb + (iota + j * _L)], flo)
                plsc.addupdate_scatter(
                    agg, [dstb + (iota + (_D // 2 + j * _L))], fhi)

    def _outer(b, c):
        pltpu.sync_copy(srcs_hbm.at[wid, pl.ds(b * _LB, _LB)], sbuf)
        pltpu.sync_copy(dsts_hbm.at[wid, pl.ds(b * _LB, _LB)], dbuf)
        rem = jnp.minimum(cnt - b * _LB, _LB)
        ng = (rem + _G - 1) // _G
        pltpu.async_copy(g_hbm.at[sbuf.at[pl.ds(0, _G)]], rows0, sem)

        def _gath(g, c2):
            @pl.when(g % 2 == 0)
            def _():
                pltpu.make_async_copy(
                    g_hbm.at[sbuf.at[pl.ds(0, _G)]], rows0, sem).wait()

                @pl.when(g + 1 < ng)
                def _():
                    pltpu.async_copy(
                        g_hbm.at[sbuf.at[pl.ds((g + 1) * _G, _G)]], rows1,
                        sem)
                _accum(rows0, g)

            @pl.when(g % 2 == 1)
            def _():
                pltpu.make_async_copy(
                    g_hbm.at[sbuf.at[pl.ds(0, _G)]], rows1, sem).wait()

                @pl.when(g + 1 < ng)
                def _():
                    pltpu.async_copy(
                        g_hbm.at[sbuf.at[pl.ds((g + 1) * _G, _G)]], rows0,
                        sem)
                _accum(rows1, g)
            return c2
        lax.fori_loop(0, ng, _gath, 0)
        return c
    lax.fori_loop(0, nouter, _outer, 0)

    off = 0
    for nrows in (64,) * (_NP // 64):
        pltpu.sync_copy(h_hbm.at[pl.ds(lo + off, nrows)],
                        hstage.at[pl.ds(0, nrows)])

        @plsc.parallel_loop(0, nrows, unroll=2)
        def _fin(i):
            dv = plsc.load_gather(dislb, [jnp.full((_L,), off + i,
                                                   jnp.int32)])
            for j in range(_D // 2 // _L):
                sl = pl.ds(j * _L, _L)
                sh = pl.ds(_D // 2 + j * _L, _L)
                a0 = agg[pl.ds((off + i) * _D + j * _L, _L)]
                a1 = agg[pl.ds((off + i) * _D + _D // 2 + j * _L, _L)]
                hn0 = -dv * a0 - hstage[i, sl]
                hn1 = -dv * a1 - hstage[i, sh]
                hstage[i, sl] = hn0
                hstage[i, sh] = hn1
                if emit_g:
                    i0 = plsc.bitcast(dv * hn0, jnp.int32) + half
                    i1 = plsc.bitcast(dv * hn1, jnp.int32) + half
                    rows0[i, sl] = ((i1 & hmask)
                                    | lax.shift_right_logical(i0, 16))
        pltpu.sync_copy(hstage.at[pl.ds(0, nrows)],
                        hn_hbm.at[pl.ds(lo + off, nrows)])
        if emit_g:
            pltpu.sync_copy(rows0.at[pl.ds(0, nrows)],
                            gn_hbm.at[pl.ds(lo + off, nrows)])
        off += nrows


_prop_scratch = [
    pltpu.VMEM(((_NP + 1) * _D,), jnp.float32),
    pltpu.VMEM((_G, _D // 2), jnp.int32),
    pltpu.VMEM((_G, _D // 2), jnp.int32),
    pltpu.VMEM((64, _D), jnp.float32),
    pltpu.VMEM((_LB,), jnp.int32),
    pltpu.VMEM((_LB,), jnp.int32),
    pltpu.VMEM((128,), jnp.int32),
    pltpu.VMEM((_NP,), jnp.float32),
    pltpu.SemaphoreType.DMA,
]

_propagate = functools.partial(
    pl.kernel,
    out_type=(
        jax.ShapeDtypeStruct((_NPAD, _D), jnp.float32),
        jax.ShapeDtypeStruct((_NPAD, _D // 2), jnp.int32),
    ),
    mesh=_sc_mesh,
    compiler_params=pltpu.CompilerParams(needs_layout_passes=False),
    scratch_types=_prop_scratch,
)(functools.partial(_propagate_body, True))

_propagate_last = functools.partial(
    pl.kernel,
    out_type=jax.ShapeDtypeStruct((_NPAD, _D), jnp.float32),
    mesh=_sc_mesh,
    compiler_params=pltpu.CompilerParams(needs_layout_passes=False),
    scratch_types=_prop_scratch,
)(functools.partial(_propagate_body, False))


def kernel(x, edge_index):
    row = edge_index[0]
    col = edge_index[1]
    srcs, dsts, cnts, dis = _edge_prep(row, col)
    xp = jnp.concatenate(
        [x, jnp.zeros((_NPAD - _N, _D), jnp.float32)], axis=0)
    h, g = _row_normalize(xp, dis)
    lxs = [h]
    for k in range(_K):
        if k < _K - 1:
            h, g = _propagate(h, g, srcs, dsts, cnts, dis)
        else:
            h = _propagate_last(h, g, srcs, dsts, cnts, dis)
        lxs.append(h)
    return _col_normalize(lxs)


# speedup vs baseline: 1.2594x; 1.0449x over previous
import functools

import jax
import jax.numpy as jnp
from jax import lax
from jax.experimental import pallas as pl
from jax.experimental.pallas import tpu as pltpu
from jax.experimental.pallas import tpu_sc as plsc

_N = 10000
_D = 256
_E = 160000
_K = 4

_NC = 2
_NS = 16
_NW = _NC * _NS
_L = 16

_NP = 320
_NPAD = _NW * _NP
_NDEG = 10240
_SEG = _NDEG // _NS
_CAP = 8192
_ECHUNK = 4000
_G = 64
_LB = 2048

_ROW_TILE = 1000


def _pack_bf16_tc(lo, hi):
    ilo = lax.bitcast_convert_type(lo, jnp.int32)
    ihi = lax.bitcast_convert_type(hi, jnp.int32)
    wlo = lax.shift_right_logical(ilo + 0x8000, 16)
    whi = (ihi + 0x8000) & (-65536)
    return whi | wlo


def _rownorm_body(x_ref, d_ref, o_ref, g_ref):
    x = x_ref[...]
    n = jnp.sqrt(jnp.sum(x * x, axis=1, keepdims=True))
    h = x / jnp.maximum(n, 1e-12)
    o_ref[...] = h
    g = h * d_ref[...]
    g_ref[...] = _pack_bf16_tc(g[:, : _D // 2], g[:, _D // 2 :])


def _colsumsq_body(h0, h1, h2, h3, h4, o_ref):
    @pl.when(pl.program_id(0) == 0)
    def _():
        o_ref[...] = jnp.zeros_like(o_ref)
    for i, h in enumerate((h0, h1, h2, h3, h4)):
        x = h[...]
        o_ref[i, :, :] += jnp.sum(x * x, axis=0, keepdims=True)


def _colscale_body(h0, h1, h2, h3, h4, s_ref, o_ref):
    for i, h in enumerate((h0, h1, h2, h3, h4)):
        o_ref[i, :, :] = h[...] * s_ref[i]


def _row_normalize(xp, dis):
    return pl.pallas_call(
        _rownorm_body,
        grid=(_NPAD // 1024,),
        in_specs=[pl.BlockSpec((1024, _D), lambda i: (i, 0)),
                  pl.BlockSpec((1024, 1), lambda i: (i, 0))],
        out_specs=(pl.BlockSpec((1024, _D), lambda i: (i, 0)),
                   pl.BlockSpec((1024, _D // 2), lambda i: (i, 0))),
        out_shape=(jax.ShapeDtypeStruct((_NPAD, _D), jnp.float32),
                   jax.ShapeDtypeStruct((_NPAD, _D // 2), jnp.int32)),
    )(xp, dis.reshape(_NPAD, 1))


def _col_normalize(lxs):
    hspec = pl.BlockSpec((_ROW_TILE, _D), lambda j: (j, 0))
    sumsq = pl.pallas_call(
        _colsumsq_body,
        grid=(_N // _ROW_TILE,),
        in_specs=[hspec] * 5,
        out_specs=pl.BlockSpec((5, 1, _D), lambda j: (0, 0, 0)),
        out_shape=jax.ShapeDtypeStruct((5, 1, _D), jnp.float32),
    )(*lxs)
    scale = 1.0 / jnp.maximum(jnp.sqrt(sumsq), 1e-12)
    return pl.pallas_call(
        _colscale_body,
        grid=(_N // _ROW_TILE,),
        in_specs=[hspec] * 5 + [pl.BlockSpec((5, 1, _D), lambda j: (0, 0, 0))],
        out_specs=pl.BlockSpec((5, _ROW_TILE, _D), lambda j: (0, j, 0)),
        out_shape=jax.ShapeDtypeStruct((5, _N, _D), jnp.float32),
    )(*lxs, scale)


_sc_mesh = plsc.VectorSubcoreMesh(
    core_axis_name="c", subcore_axis_name="s",
    num_cores=_NC, num_subcores=_NS)


def _edge_prep_body(row_hbm, col_hbm, srcs_hbm, dsts_hbm, cnts_hbm, dis_hbm,
                    degb, tmpb, tmpb2, srcl, dstl, rbuf0, cbuf0, rbuf1, cbuf1,
                    cntb, dislb, shdeg, shdis, sem0, sem1):
    cid = lax.axis_index("c")
    sid = lax.axis_index("s")
    wid = cid * _NS + sid
    lo = wid * _NP
    iota = lax.iota(jnp.int32, _L)
    zf = jnp.zeros((_L,), jnp.float32)
    zi = jnp.zeros((_L,), jnp.int32)

    def _z(i, c):
        degb[pl.ds(i * _L, _L)] = zf
        return c
    lax.fori_loop(0, _NDEG // _L, _z, 0)

    ebase = sid * (_E // _NS)
    ec1 = 2000
    for ch in range(_E // _NS // ec1):
        pltpu.sync_copy(row_hbm.at[pl.ds(ebase + ch * ec1, ec1)],
                        rbuf0.at[pl.ds(0, ec1)])
        pltpu.sync_copy(col_hbm.at[pl.ds(ebase + ch * ec1, ec1)],
                        cbuf0.at[pl.ds(0, ec1)])

        def _deg(v, c):
            r = rbuf0[pl.ds(v * _L, _L)]
            cc = cbuf0[pl.ds(v * _L, _L)]
            w = jnp.where(r != cc, 1.0, 0.0).astype(jnp.float32)
            plsc.addupdate_scatter(degb, [r], w)
            return c
        lax.fori_loop(0, ec1 // _L, _deg, 0)

    pltpu.sync_copy(degb, shdeg.at[sid])
    plsc.subcore_barrier()
    pltpu.sync_copy(shdeg.at[0, pl.ds(sid * _SEG, _SEG)], tmpb)
    for t in range(1, _NS):
        pltpu.sync_copy(shdeg.at[t, pl.ds(sid * _SEG, _SEG)], tmpb2)

        def _acc(v, c):
            s = pl.ds(v * _L, _L)
            tmpb[s] = tmpb[s] + tmpb2[s]
            return c
        lax.fori_loop(0, _SEG // _L, _acc, 0)

    def _newton(v, c):
        s = pl.ds(v * _L, _L)
        d = tmpb[s]
        x = jnp.maximum(d, 1.0)
        i = plsc.bitcast(x, jnp.int32)
        i = jnp.int32(0x5F3759DF) - lax.shift_right_logical(i, 1)
        y = plsc.bitcast(i, jnp.float32)
        for _ in range(3):
            y = y * (1.5 - 0.5 * x * y * y)
        tmpb[s] = jnp.where(d > 0.0, y, 0.0)
        return c
    lax.fori_loop(0, _SEG // _L, _newton, 0)
    pltpu.sync_copy(tmpb, shdis.at[pl.ds(sid * _SEG, _SEG)])
    plsc.subcore_barrier()

    pltpu.sync_copy(shdis.at[pl.ds(lo, _NP)], dislb)
    pltpu.sync_copy(dislb, dis_hbm.at[pl.ds(lo, _NP)])

    def _scan_buf(rbuf, cbuf, basev):
        @plsc.parallel_loop(0, _ECHUNK // _L, unroll=2, carry=basev)
        def _scan(v, bv):
            r = rbuf[pl.ds(v * _L, _L)]
            cc = cbuf[pl.ds(v * _L, _L)]
            m = (cc >= lo) & (cc < lo + _NP) & (r != cc)
            mi = jnp.where(m, 1, 0).astype(jnp.int32)
            cs = plsc.cumsum(mi)
            pos = bv + cs - mi
            mok = m & (pos < _CAP)
            plsc.store_scatter(srcl, [pos], r, mask=mok)
            plsc.store_scatter(dstl, [pos],
                               lax.shift_left(cc - lo, 8), mask=mok)
            return jnp.minimum(bv + jnp.max(cs), _CAP)
        return _scan

    nch = _E // _ECHUNK
    pltpu.async_copy(row_hbm.at[pl.ds(0, _ECHUNK)], rbuf0, sem0)
    pltpu.async_copy(col_hbm.at[pl.ds(0, _ECHUNK)], cbuf0, sem0)

    def _pair(p, basev):
        c0 = 2 * p
        pltpu.make_async_copy(
            row_hbm.at[pl.ds(c0 * _ECHUNK, _ECHUNK)], rbuf0, sem0).wait()
        pltpu.make_async_copy(
            col_hbm.at[pl.ds(c0 * _ECHUNK, _ECHUNK)], cbuf0, sem0).wait()
        pltpu.async_copy(
            row_hbm.at[pl.ds((c0 + 1) * _ECHUNK, _ECHUNK)], rbuf1, sem1)
        pltpu.async_copy(
            col_hbm.at[pl.ds((c0 + 1) * _ECHUNK, _ECHUNK)], cbuf1, sem1)
        basev = _scan_buf(rbuf0, cbuf0, basev)
        pltpu.make_async_copy(
            row_hbm.at[pl.ds((c0 + 1) * _ECHUNK, _ECHUNK)], rbuf1,
            sem1).wait()
        pltpu.make_async_copy(
            col_hbm.at[pl.ds((c0 + 1) * _ECHUNK, _ECHUNK)], cbuf1,
            sem1).wait()

        @pl.when(p + 1 < nch // 2)
        def _():
            pltpu.async_copy(
                row_hbm.at[pl.ds((c0 + 2) * _ECHUNK, _ECHUNK)], rbuf0, sem0)
            pltpu.async_copy(
                col_hbm.at[pl.ds((c0 + 2) * _ECHUNK, _ECHUNK)], cbuf0, sem0)
        basev = _scan_buf(rbuf1, cbuf1, basev)
        return basev

    basev = lax.fori_loop(0, nch // 2, _pair, jnp.zeros((_L,), jnp.int32))

    trash = jnp.full((_L,), _NP * _D, jnp.int32)
    for j in range(_G // _L):
        pos = basev + j * _L + iota
        mok = pos < _CAP
        plsc.store_scatter(srcl, [pos], zi, mask=mok)
        plsc.store_scatter(dstl, [pos], trash, mask=mok)

    cnt = jnp.minimum(basev, _CAP - _G)
    for j in range(128 // _L):
        cntb[pl.ds(j * _L, _L)] = cnt
    pltpu.sync_copy(cntb, cnts_hbm.at[wid])
    pltpu.sync_copy(srcl, srcs_hbm.at[wid])
    pltpu.sync_copy(dstl, dsts_hbm.at[wid])


_edge_prep = functools.partial(
    pl.kernel,
    out_type=(
        jax.ShapeDtypeStruct((_NW, _CAP), jnp.int32),
        jax.ShapeDtypeStruct((_NW, _CAP), jnp.int32),
        jax.ShapeDtypeStruct((_NW, 128), jnp.int32),
        jax.ShapeDtypeStruct((_NPAD,), jnp.float32),
    ),
    mesh=_sc_mesh,
    compiler_params=pltpu.CompilerParams(needs_layout_passes=False),
    scratch_types=[
        pltpu.VMEM((_NDEG,), jnp.float32),
        pltpu.VMEM((_SEG,), jnp.float32),
        pltpu.VMEM((_SEG,), jnp.float32),
        pltpu.VMEM((_CAP,), jnp.int32),
        pltpu.VMEM((_CAP,), jnp.int32),
        pltpu.VMEM((_ECHUNK,), jnp.int32),
        pltpu.VMEM((_ECHUNK,), jnp.int32),
        pltpu.VMEM((_ECHUNK,), jnp.int32),
        pltpu.VMEM((_ECHUNK,), jnp.int32),
        pltpu.VMEM((128,), jnp.int32),
        pltpu.VMEM((_NP,), jnp.float32),
        pltpu.VMEM_SHARED((_NS, _NDEG), jnp.float32),
        pltpu.VMEM_SHARED((_NDEG,), jnp.float32),
        pltpu.SemaphoreType.DMA,
        pltpu.SemaphoreType.DMA,
    ],
)(_edge_prep_body)


def _propagate_body(emit_g, h_hbm, g_hbm, srcs_hbm, dsts_hbm, cnts_hbm,
                    dis_hbm, *refs):
    if emit_g:
        (hn_hbm, gn_hbm, agg, rows0, rows1, hstage, sbuf, dbuf, cbuf, dislb,
         sem) = refs
    else:
        (hn_hbm, agg, rows0, rows1, hstage, sbuf, dbuf, cbuf, dislb,
         sem) = refs
        gn_hbm = None
    cid = lax.axis_index("c")
    sid = lax.axis_index("s")
    wid = cid * _NS + sid
    lo = wid * _NP
    iota = lax.iota(jnp.int32, _L)
    zf = jnp.zeros((_L,), jnp.float32)
    hmask = jnp.full((_L,), -65536, jnp.int32)
    half = jnp.full((_L,), 0x8000, jnp.int32)

    @plsc.parallel_loop(0, (_NP + 1) * _D // _L, unroll=4)
    def _z(i):
        agg[pl.ds(i * _L, _L)] = zf

    pltpu.sync_copy(cnts_hbm.at[wid], cbuf)
    pltpu.sync_copy(dis_hbm.at[pl.ds(lo, _NP)], dislb)
    cnt = jnp.max(cbuf[pl.ds(0, _L)])
    nouter = (cnt + _LB - 1) // _LB

    def _accum(rbufref, g):
        @plsc.parallel_loop(0, _G, unroll=2)
        def _e(e):
            v = g * _G + e
            vf = jnp.full((_L,), v, jnp.int32)
            dstb = plsc.load_gather(dbuf, [vf])
            for j in range(_D // 2 // _L):
                w = rbufref[e, pl.ds(j * _L, _L)]
                flo = plsc.bitcast(lax.shift_left(w, 16), jnp.float32)
                fhi = plsc.bitcast(w & hmask, jnp.float32)
                plsc.addupdate_scatter(
                    agg, [dstb + (iota + j * _L)], flo)
                plsc.addupdate_scatter(
                    agg, [dstb + (iota + (_D // 2 + j * _L))], fhi)

    def _outer(b, c):
        pltpu.sync_copy(srcs_hbm.at[wid, pl.ds(b * _LB, _LB)], sbuf)
        pltpu.sync_copy(dsts_hbm.at[wid, pl.ds(b * _LB, _LB)], dbuf)
        rem = jnp.minimum(cnt - b * _LB, _LB)
        ng = (rem + _G - 1) // _G
        pltpu.async_copy(g_hbm.at[sbuf.at[pl.ds(0, _G)]], rows0, sem)

        def _gath(g, c2):
            @pl.when(g % 2 == 0)
            def _():
                pltpu.make_async_copy(
                    g_hbm.at[sbuf.at[pl.ds(0, _G)]], rows0, sem).wait()

                @pl.when(g + 1 < ng)
                def _():
                    pltpu.async_copy(
                        g_hbm.at[sbuf.at[pl.ds((g + 1) * _G, _G)]], rows1,
                        sem)
                _accum(rows0, g)

            @pl.when(g % 2 == 1)
            def _():
                pltpu.make_async_copy(
                    g_hbm.at[sbuf.at[pl.ds(0, _G)]], rows1, sem).wait()

                @pl.when(g + 1 < ng)
                def _():
                    pltpu.async_copy(
                        g_hbm.at[sbuf.at[pl.ds((g + 1) * _G, _G)]], rows0,
                        sem)
                _accum(rows1, g)
            return c2
        lax.fori_loop(0, ng, _gath, 0)
        return c
    lax.fori_loop(0, nouter, _outer, 0)

    off = 0
    for nrows in (64,) * (_NP // 64):
        pltpu.sync_copy(h_hbm.at[pl.ds(lo + off, nrows)],
                        hstage.at[pl.ds(0, nrows)])

        @plsc.parallel_loop(0, nrows, unroll=2)
        def _fin(i):
            dv = plsc.load_gather(dislb, [jnp.full((_L,), off + i,
                                                   jnp.int32)])
            for j in range(_D // 2 // _L):
                sl = pl.ds(j * _L, _L)
                sh = pl.ds(_D // 2 + j * _L, _L)
                a0 = agg[pl.ds((off + i) * _D + j * _L, _L)]
                a1 = agg[pl.ds((off + i) * _D + _D // 2 + j * _L, _L)]
                hn0 = -dv * a0 - hstage[i, sl]
                hn1 = -dv * a1 - hstage[i, sh]
                hstage[i, sl] = hn0
                hstage[i, sh] = hn1
                if emit_g:
                    i0 = plsc.bitcast(dv * hn0, jnp.int32) + half
                    i1 = plsc.bitcast(dv * hn1, jnp.int32) + half
                    rows0[i, sl] = ((i1 & hmask)
                                    | lax.shift_right_logical(i0, 16))
        pltpu.sync_copy(hstage.at[pl.ds(0, nrows)],
                        hn_hbm.at[pl.ds(lo + off, nrows)])
        if emit_g:
            pltpu.sync_copy(rows0.at[pl.ds(0, nrows)],
                            gn_hbm.at[pl.ds(lo + off, nrows)])
        off += nrows


_prop_scratch = [
    pltpu.VMEM(((_NP + 1) * _D,), jnp.float32),
    pltpu.VMEM((_G, _D // 2), jnp.int32),
    pltpu.VMEM((_G, _D // 2), jnp.int32),
    pltpu.VMEM((64, _D), jnp.float32),
    pltpu.VMEM((_LB,), jnp.int32),
    pltpu.VMEM((_LB,), jnp.int32),
    pltpu.VMEM((128,), jnp.int32),
    pltpu.VMEM((_NP,), jnp.float32),
    pltpu.SemaphoreType.DMA,
]

_propagate = functools.partial(
    pl.kernel,
    out_type=(
        jax.ShapeDtypeStruct((_NPAD, _D), jnp.float32),
        jax.ShapeDtypeStruct((_NPAD, _D // 2), jnp.int32),
    ),
    mesh=_sc_mesh,
    compiler_params=pltpu.CompilerParams(needs_layout_passes=False),
    scratch_types=_prop_scratch,
)(functools.partial(_propagate_body, True))

_propagate_last = functools.partial(
    pl.kernel,
    out_type=jax.ShapeDtypeStruct((_NPAD, _D), jnp.float32),
    mesh=_sc_mesh,
    compiler_params=pltpu.CompilerParams(needs_layout_passes=False),
    scratch_types=_prop_scratch,
)(functools.partial(_propagate_body, False))


def kernel(x, edge_index):
    row = edge_index[0]
    col = edge_index[1]
    srcs, dsts, cnts, dis = _edge_prep(row, col)
    xp = jnp.concatenate(
        [x, jnp.zeros((_NPAD - _N, _D), jnp.float32)], axis=0)
    h, g = _row_normalize(xp, dis)
    lxs = [h]
    for k in range(_K):
        if k < _K - 1:
            h, g = _propagate(h, g, srcs, dsts, cnts, dis)
        else:
            h = _propagate_last(h, g, srcs, dsts, cnts, dis)
        lxs.append(h)
    return _col_normalize(lxs)
